# baseline (speedup 1.0000x reference)
import jax
import jax.numpy as jnp
from jax.experimental import pallas as pl
from jax.experimental.pallas import tpu as pltpu
from jax.experimental.pallas import tpu_sc as plsc

import functools


@functools.partial(jax.jit, static_argnames=("batch", "seq_len", "hidden"))
def _gather_last(sequences, lengths, *, batch, seq_len, hidden):
    half = hidden // 2
    mesh = plsc.ScalarSubcoreMesh(axis_name="c", num_cores=1)

    @functools.partial(
        pl.kernel,
        out_type=jax.ShapeDtypeStruct((batch, 1, hidden), jnp.float32),
        mesh=mesh,
        scratch_types=[
            pltpu.SMEM((batch,), jnp.int32),
            pltpu.SemaphoreType.DMA,
        ],
    )
    def k(seq_hbm, len_hbm, out_hbm, len_sm, sem):
        copies = [
            pltpu.async_copy(
                seq_hbm.at[:, pl.ds(0, 1), pl.ds(half, half)],
                out_hbm.at[:, :, pl.ds(half, half)],
                sem,
            ),
            pltpu.async_copy(len_hbm, len_sm, sem),
        ]
        copies[1].wait()
        copies.pop()
        for b in range(batch):
            t = len_sm[b] - 1
            copies.append(
                pltpu.async_copy(
                    seq_hbm.at[pl.ds(b, 1), pl.ds(t, 1), pl.ds(0, half)],
                    out_hbm.at[pl.ds(b, 1), :, pl.ds(0, half)],
                    sem,
                )
            )
        for c in copies:
            c.wait()

    return k(sequences, lengths)


def kernel(sequences, lengths):
    batch, seq_len, hidden_x_dirs = sequences.shape
    out = _gather_last(
        sequences,
        lengths.astype(jnp.int32),
        batch=batch,
        seq_len=seq_len,
        hidden=hidden_x_dirs,
    )
    return out.reshape(batch, hidden_x_dirs)


# speedup vs baseline: 1.0538x; 1.0538x over previous
import jax
import jax.numpy as jnp
from jax import lax
from jax.experimental import pallas as pl
from jax.experimental.pallas import tpu as pltpu
from jax.experimental.pallas import tpu_sc as plsc

import functools

_NC = 2


@functools.partial(jax.jit, static_argnames=("batch", "seq_len", "hidden"))
def _gather_last(seq_rows, lengths, *, batch, seq_len, hidden):
    half = hidden // 2
    mesh = plsc.ScalarSubcoreMesh(axis_name="c", num_cores=1)

    @functools.partial(
        pl.kernel,
        out_type=jax.ShapeDtypeStruct((batch, hidden), jnp.float32),
        mesh=mesh,
        scratch_types=[
            pltpu.SMEM((batch,), jnp.int32),
            pltpu.SemaphoreType.DMA,
        ],
    )
    def k(seq_hbm, len_hbm, out_hbm, len_sm, sem):
        pltpu.sync_copy(len_hbm, len_sm)
        copies = []
        for b in range(batch):
            row = b * seq_len + len_sm[b] - 1
            copies.append(
                pltpu.async_copy(
                    seq_hbm.at[pl.ds(row, 1), pl.ds(0, half)],
                    out_hbm.at[pl.ds(b, 1), pl.ds(0, half)],
                    sem,
                )
            )
            copies.append(
                pltpu.async_copy(
                    seq_hbm.at[pl.ds(b * seq_len, 1), pl.ds(half, half)],
                    out_hbm.at[pl.ds(b, 1), pl.ds(half, half)],
                    sem,
                )
            )
        for c in copies:
            c.wait()

    return k(seq_rows, lengths)


def kernel(sequences, lengths):
    batch, seq_len, hidden_x_dirs = sequences.shape
    seq_rows = sequences.reshape(batch * seq_len, hidden_x_dirs)
    return _gather_last(
        seq_rows,
        lengths.astype(jnp.int32),
        batch=batch,
        seq_len=seq_len,
        hidden=hidden_x_dirs,
    )


# speedup vs baseline: 1.0565x; 1.0025x over previous
import jax
import jax.numpy as jnp
from jax import lax
from jax.experimental import pallas as pl
from jax.experimental.pallas import tpu as pltpu
from jax.experimental.pallas import tpu_sc as plsc

import functools

_NC = 2


@functools.partial(jax.jit, static_argnames=("batch", "seq_len", "hidden"))
def _gather_last(seq_rows, lengths, *, batch, seq_len, hidden):
    half = hidden // 2
    mesh = plsc.ScalarSubcoreMesh(axis_name="c", num_cores=1)

    @functools.partial(
        pl.kernel,
        out_type=jax.ShapeDtypeStruct((batch, hidden), jnp.float32),
        mesh=mesh,
        scratch_types=[
            pltpu.SMEM((batch,), jnp.int32),
            pltpu.SemaphoreType.DMA,
        ],
    )
    def k(seq_hbm, len_hbm, out_hbm, len_sm, sem):
        pltpu.sync_copy(len_hbm, len_sm)

        def issue(b, carry):
            row = b * seq_len + len_sm[b] - 1
            pltpu.async_copy(
                seq_hbm.at[pl.ds(row, 1), pl.ds(0, half)],
                out_hbm.at[pl.ds(b, 1), pl.ds(0, half)],
                sem,
            )
            pltpu.async_copy(
                seq_hbm.at[pl.ds(b * seq_len, 1), pl.ds(half, half)],
                out_hbm.at[pl.ds(b, 1), pl.ds(half, half)],
                sem,
            )
            return carry

        lax.fori_loop(0, batch, issue, 0)

        def drain(_, carry):
            pltpu.make_async_copy(
                seq_hbm.at[pl.ds(0, 1), pl.ds(0, half)],
                out_hbm.at[pl.ds(0, 1), pl.ds(0, half)],
                sem,
            ).wait()
            return carry

        lax.fori_loop(0, 2 * batch, drain, 0)

    return k(seq_rows, lengths)


def kernel(sequences, lengths):
    batch, seq_len, hidden_x_dirs = sequences.shape
    seq_rows = sequences.reshape(batch * seq_len, hidden_x_dirs)
    return _gather_last(
        seq_rows,
        lengths.astype(jnp.int32),
        batch=batch,
        seq_len=seq_len,
        hidden=hidden_x_dirs,
    )


# speedup vs baseline: 1.0760x; 1.0185x over previous
import jax
import jax.numpy as jnp
from jax import lax
from jax.experimental import pallas as pl
from jax.experimental.pallas import tpu as pltpu
from jax.experimental.pallas import tpu_sc as plsc

import functools


@functools.partial(jax.jit, static_argnames=("batch", "seq_len", "hidden"))
def _gather_last(seq3, lengths, *, batch, seq_len, hidden):
    half = hidden // 2
    mesh = plsc.ScalarSubcoreMesh(axis_name="c", num_cores=1)

    @functools.partial(
        pl.kernel,
        out_type=jax.ShapeDtypeStruct((batch, hidden), jnp.float32),
        mesh=mesh,
        scratch_types=[
            pltpu.SMEM((batch,), jnp.int32),
            pltpu.SemaphoreType.DMA,
        ],
    )
    def k(seq3_hbm, len_hbm, out_hbm, len_sm, sem):
        bw = pltpu.async_copy(
            seq3_hbm.at[:, 0, pl.ds(half, half)],
            out_hbm.at[:, pl.ds(half, half)],
            sem,
        )
        pltpu.sync_copy(len_hbm, len_sm)
        copies = []
        for b in range(batch):
            t = len_sm[b] - 1
            copies.append(
                pltpu.async_copy(
                    seq3_hbm.at[b, pl.ds(t, 1), pl.ds(0, half)],
                    out_hbm.at[pl.ds(b, 1), pl.ds(0, half)],
                    sem,
                )
            )
        bw.wait()
        for c in copies:
            c.wait()

    return k(seq3, lengths)


def kernel(sequences, lengths):
    batch, seq_len, hidden_x_dirs = sequences.shape
    return _gather_last(
        sequences,
        lengths.astype(jnp.int32),
        batch=batch,
        seq_len=seq_len,
        hidden=hidden_x_dirs,
    )
